# baseline (speedup 1.0000x reference)
import numpy as np
import jax
import jax.numpy as jnp
from jax.experimental import pallas as pl
from jax.experimental.pallas import tpu as pltpu

_L = 2048
_TOPK = 7
_BN = 256
_BM = 256
_NBLK = _L // _BN


def _build_dft_consts():
    t = np.arange(_L // 2, dtype=np.float64)
    om = np.arange(1, _L // 2 + 1, dtype=np.float64)
    th = 2.0 * np.pi * np.outer(om, t) / _L
    gr = np.cos(th)
    gi = -np.sin(th)
    return gr.astype(np.float32), gi.astype(np.float32)


def _split_hi_lo(a):
    import ml_dtypes
    hi32 = a.astype(ml_dtypes.bfloat16).astype(np.float32)
    lo32 = a - hi32
    return hi32, lo32


_GR_NP, _GI_NP = _build_dft_consts()
_GRH, _GRL = _split_hi_lo(_GR_NP)
_GIH, _GIL = _split_hi_lo(_GI_NP)


def _split_f32(x):
    hi = x.astype(jnp.bfloat16)
    lo = (x - hi.astype(jnp.float32)).astype(jnp.bfloat16)
    return hi, lo


def _dot3(ah, al, bh, bl):
    acc = jnp.dot(ah, bh, preferred_element_type=jnp.float32)
    acc += jnp.dot(ah, bl, preferred_element_type=jnp.float32)
    acc += jnp.dot(al, bh, preferred_element_type=jnp.float32)
    return acc


def _dot3_t(ah, al, bh, bl):
    dn = (((0,), (0,)), ((), ()))
    acc = jax.lax.dot_general(ah, bh, dn, preferred_element_type=jnp.float32)
    acc += jax.lax.dot_general(ah, bl, dn, preferred_element_type=jnp.float32)
    acc += jax.lax.dot_general(al, bh, dn, preferred_element_type=jnp.float32)
    return acc


def _fold_spectrum(grh_ref, grl_ref, gih_ref, gil_ref, xb, xrev):
    h = _L // 2
    iot_h = jax.lax.broadcasted_iota(jnp.int32, (h, _BN), 0)
    xe = jnp.where(iot_h == 0, xb[0:1, :], xb[:h] + xrev[:h])
    xo = xb[:h] - xrev[:h]
    xeh, xel = _split_f32(xe)
    xoh, xol = _split_f32(xo)
    alt = jnp.where(jax.lax.rem(iot_h, 2) == 0, jnp.float32(-1.0),
                    jnp.float32(1.0))
    xr = _dot3(grh_ref[...], grl_ref[...], xeh, xel)
    xr = xr + alt * xb[h:h + 1, :]
    xi = _dot3(gih_ref[...], gil_ref[...], xoh, xol)
    return xr, xi, alt


def _synth(grh_ref, grl_ref, gih_ref, gil_ref, qb, qrev, kb, krev,
           corr_ref, dc_ref):
    h = _L // 2
    qr, qi, alt = _fold_spectrum(grh_ref, grl_ref, gih_ref, gil_ref,
                                 qb, qrev)
    kr, ki, _ = _fold_spectrum(grh_ref, grl_ref, gih_ref, gil_ref,
                               kb, krev)
    iot_h = jax.lax.broadcasted_iota(jnp.int32, (h, _BN), 0)
    sc_re = jnp.where(iot_h == h - 1, jnp.float32(1.0 / _L),
                      jnp.float32(2.0 / _L))
    yre = (qr * kr + qi * ki) * sc_re
    yim = (qi * kr - qr * ki) * jnp.float32(2.0 / _L)
    yrh, yrl = _split_f32(yre)
    yih, yil = _split_f32(yim)
    acos = _dot3_t(grh_ref[...], grl_ref[...], yrh, yrl)
    bsin = _dot3_t(gih_ref[...], gil_ref[...], yih, yil)
    corr_lo = acos + bsin
    c1024 = jnp.sum(alt * yre, axis=0, keepdims=True)
    hi = acos - bsin
    corr_hi = jnp.where(iot_h == 0, jnp.broadcast_to(c1024, (h, _BN)), hi)
    corr_ref[0:h, :] = corr_lo
    corr_ref[h:_L, :] = corr_hi
    qs = jnp.sum(qb, axis=0, keepdims=True)
    ks = jnp.sum(kb, axis=0, keepdims=True)
    dc_ref[...] = jnp.broadcast_to(qs * ks * (1.0 / _L), (8, _BN))


def _sub8(op, x8, shift):
    return op(x8, pltpu.roll(x8, shift, axis=0))


def _red8(op, x3):
    x8 = op.reduce(x3, axis=0)
    x8 = _sub8(op.combine, x8, 4)
    x8 = _sub8(op.combine, x8, 2)
    x8 = _sub8(op.combine, x8, 1)
    return x8


class _Max:
    reduce = staticmethod(lambda x, axis: jnp.max(x, axis=axis))
    combine = staticmethod(jnp.maximum)


class _Min:
    reduce = staticmethod(lambda x, axis: jnp.min(x, axis=axis))
    combine = staticmethod(jnp.minimum)


class _Sum:
    reduce = staticmethod(lambda x, axis: jnp.sum(x, axis=axis))
    combine = staticmethod(lambda a, b: a + b)


def _topk(corr_ref, dc_ref, vb, w_ref, p_ref):
    g = _L // 8
    h = _L // 2
    dc = dc_ref[0:1, :]
    corr3 = corr_ref[...].reshape(g, 8, _BN)
    vb3 = vb.reshape(g, 8, _BN)
    rr = (jax.lax.broadcasted_iota(jnp.int32, (g, 8, _BN), 0) * 8
          + jax.lax.broadcasted_iota(jnp.int32, (g, 8, _BN), 1))
    iot3 = jnp.where(rr < h, rr, 3 * h - rr)
    iot3 = jnp.where(rr == h, h, iot3)
    wrows = []
    prows = []
    neg = jnp.float32(-jnp.inf)
    m = _red8(_Max, corr3)
    for i in range(_TOPK):
        idx = _red8(_Min, jnp.where(corr3 == m[None], iot3, _L))
        sel = iot3 == idx[None]
        pat = _red8(_Sum, jnp.where(sel, vb3, 0.0))
        wrows.append(m[0:1] + dc)
        prows.append(pat[0:1])
        if i + 1 < _TOPK:
            corr3 = jnp.where(sel, neg, corr3)
            m = _red8(_Max, corr3)
    zero = jnp.zeros_like(wrows[0])
    w_ref[...] = jnp.concatenate(wrows + [zero], axis=0)
    p_ref[...] = jnp.concatenate(prows + [zero], axis=0)


def _main_kernel(grh_ref, grl_ref, gih_ref, gil_ref,
                 q_ref, qr_ref, k_ref, kr_ref, v_ref,
                 w_ref, p_ref, corr_scr, dc_scr):
    j = pl.program_id(0)
    parity = jax.lax.rem(j, 2)
    omp = 1 - parity
    _topk(corr_scr.at[omp], dc_scr.at[omp], v_ref[...], w_ref, p_ref)
    _synth(grh_ref, grl_ref, gih_ref, gil_ref,
           q_ref[...], qr_ref[...], k_ref[...], kr_ref[...],
           corr_scr.at[parity], dc_scr.at[parity])


def _outer_kernel(p_ref, w_ref, o_ref):
    ph, plo = _split_f32(p_ref[...])
    wh, wl = _split_f32(w_ref[...])
    o_ref[...] = _dot3_t(ph, plo, wh, wl)


def kernel(Q, K, V):
    q0 = Q[0]
    k0 = K[0]
    v0 = V[0]
    h = _L // 2
    qrev = jnp.roll(jnp.flip(q0, axis=0), 1, axis=0)
    krev = jnp.roll(jnp.flip(k0, axis=0), 1, axis=0)
    vrev = jnp.roll(jnp.flip(v0, axis=0), 1, axis=0)
    vperm = jnp.concatenate(
        [v0[:h], v0[h:h + 1], vrev[1:h]], axis=0)
    grh = jnp.asarray(_GRH).astype(jnp.bfloat16)
    grl = jnp.asarray(_GRL).astype(jnp.bfloat16)
    gih = jnp.asarray(_GIH).astype(jnp.bfloat16)
    gil = jnp.asarray(_GIL).astype(jnp.bfloat16)

    full = pl.BlockSpec((_L // 2, _L // 2), lambda j: (0, 0))
    cur = pl.BlockSpec((_L, _BN), lambda j: (0, jnp.minimum(j, _NBLK - 1)))
    lag = pl.BlockSpec((_L, _BN), lambda j: (0, jnp.maximum(j - 1, 0)))
    lag8 = pl.BlockSpec((8, _BN), lambda j: (0, jnp.maximum(j - 1, 0)))

    wt, pt = pl.pallas_call(
        _main_kernel,
        grid=(_NBLK + 1,),
        in_specs=[full, full, full, full, cur, cur, cur, cur, lag],
        out_specs=[lag8, lag8],
        out_shape=[
            jax.ShapeDtypeStruct((8, _L), jnp.float32),
            jax.ShapeDtypeStruct((8, _L), jnp.float32),
        ],
        scratch_shapes=[
            pltpu.VMEM((2, _L, _BN), jnp.float32),
            pltpu.VMEM((2, 8, _BN), jnp.float32),
        ],
        compiler_params=pltpu.CompilerParams(
            vmem_limit_bytes=64 * 1024 * 1024),
    )(grh, grl, gih, gil, q0, qrev, k0, krev, vperm)

    out = pl.pallas_call(
        _outer_kernel,
        grid=(_L // _BM,),
        in_specs=[
            pl.BlockSpec((8, _BM), lambda i: (0, i)),
            pl.BlockSpec((8, _L), lambda i: (0, 0)),
        ],
        out_specs=pl.BlockSpec((_BM, _L), lambda i: (i, 0)),
        out_shape=jax.ShapeDtypeStruct((_L, _L), jnp.float32),
    )(pt, wt)
    return out[None]


# speedup vs baseline: 1.0707x; 1.0707x over previous
import numpy as np
import jax
import jax.numpy as jnp
from jax.experimental import pallas as pl
from jax.experimental.pallas import tpu as pltpu

_L = 2048
_TOPK = 7
_BN = 256
_BM = 256
_NBLK = _L // _BN


def _build_dft_consts():
    t = np.arange(_L // 2, dtype=np.float64)
    om = np.arange(1, _L // 2 + 1, dtype=np.float64)
    th = 2.0 * np.pi * np.outer(om, t) / _L
    gr = np.cos(th)
    gi = -np.sin(th)
    return gr.astype(np.float32), gi.astype(np.float32)


def _split_hi_lo(a):
    import ml_dtypes
    hi32 = a.astype(ml_dtypes.bfloat16).astype(np.float32)
    lo32 = a - hi32
    return hi32, lo32


_GR_NP, _GI_NP = _build_dft_consts()
_GRH, _GRL = _split_hi_lo(_GR_NP)
_GIH, _GIL = _split_hi_lo(_GI_NP)


def _split_f32(x):
    hi = x.astype(jnp.bfloat16)
    lo = (x - hi.astype(jnp.float32)).astype(jnp.bfloat16)
    return hi, lo


def _dot3(ah, al, bh, bl):
    acc = jnp.dot(ah, bh, preferred_element_type=jnp.float32)
    acc += jnp.dot(ah, bl, preferred_element_type=jnp.float32)
    acc += jnp.dot(al, bh, preferred_element_type=jnp.float32)
    return acc


def _dot3_t(ah, al, bh, bl):
    dn = (((0,), (0,)), ((), ()))
    acc = jax.lax.dot_general(ah, bh, dn, preferred_element_type=jnp.float32)
    acc += jax.lax.dot_general(ah, bl, dn, preferred_element_type=jnp.float32)
    acc += jax.lax.dot_general(al, bh, dn, preferred_element_type=jnp.float32)
    return acc


def _fold_spectrum(grh_ref, grl_ref, gih_ref, gil_ref, xb, xrev):
    h = _L // 2
    iot_h = jax.lax.broadcasted_iota(jnp.int32, (h, _BN), 0)
    xe = jnp.where(iot_h == 0, xb[0:1, :], xb[:h] + xrev[:h])
    xo = xb[:h] - xrev[:h]
    xeh, xel = _split_f32(xe)
    xoh, xol = _split_f32(xo)
    alt = jnp.where(jax.lax.rem(iot_h, 2) == 0, jnp.float32(-1.0),
                    jnp.float32(1.0))
    xr = _dot3(grh_ref[...], grl_ref[...], xeh, xel)
    xr = xr + alt * xb[h:h + 1, :]
    xi = _dot3(gih_ref[...], gil_ref[...], xoh, xol)
    return xr, xi, alt


def _synth(grh_ref, grl_ref, gih_ref, gil_ref, qb, qrev, kb, krev,
           corr_ref, dc_ref):
    h = _L // 2
    qr, qi, alt = _fold_spectrum(grh_ref, grl_ref, gih_ref, gil_ref,
                                 qb, qrev)
    kr, ki, _ = _fold_spectrum(grh_ref, grl_ref, gih_ref, gil_ref,
                               kb, krev)
    iot_h = jax.lax.broadcasted_iota(jnp.int32, (h, _BN), 0)
    sc_re = jnp.where(iot_h == h - 1, jnp.float32(1.0 / _L),
                      jnp.float32(2.0 / _L))
    yre = (qr * kr + qi * ki) * sc_re
    yim = (qi * kr - qr * ki) * jnp.float32(2.0 / _L)
    yrh, yrl = _split_f32(yre)
    yih, yil = _split_f32(yim)
    acos = _dot3_t(grh_ref[...], grl_ref[...], yrh, yrl)
    bsin = _dot3_t(gih_ref[...], gil_ref[...], yih, yil)
    corr_lo = acos + bsin
    c1024 = jnp.sum(alt * yre, axis=0, keepdims=True)
    hi = acos - bsin
    corr_hi = jnp.where(iot_h == 0, jnp.broadcast_to(c1024, (h, _BN)), hi)
    corr_ref[0:h, :] = corr_lo
    corr_ref[h:_L, :] = corr_hi
    qs = jnp.sum(qb, axis=0, keepdims=True)
    ks = jnp.sum(kb, axis=0, keepdims=True)
    dc_ref[...] = jnp.broadcast_to(qs * ks * (1.0 / _L), (8, _BN))


def _sub8(op, x8, shift):
    return op(x8, pltpu.roll(x8, shift, axis=0))


def _red8(op, x3):
    x8 = op.reduce(x3, axis=0)
    x8 = _sub8(op.combine, x8, 4)
    x8 = _sub8(op.combine, x8, 2)
    x8 = _sub8(op.combine, x8, 1)
    return x8


class _Max:
    reduce = staticmethod(lambda x, axis: jnp.max(x, axis=axis))
    combine = staticmethod(jnp.maximum)


class _Min:
    reduce = staticmethod(lambda x, axis: jnp.min(x, axis=axis))
    combine = staticmethod(jnp.minimum)


class _Sum:
    reduce = staticmethod(lambda x, axis: jnp.sum(x, axis=axis))
    combine = staticmethod(lambda a, b: a + b)


def _topk(corr_ref, dc_ref, vb, w_ref, p_ref):
    g = _L // 8
    h = _L // 2
    dc = dc_ref[0:1, :]
    corr3 = corr_ref[...].reshape(g, 8, _BN)
    vb3 = vb.reshape(g, 8, _BN)
    rr = (jax.lax.broadcasted_iota(jnp.int32, (g, 8, _BN), 0) * 8
          + jax.lax.broadcasted_iota(jnp.int32, (g, 8, _BN), 1))
    iot3 = jnp.where(rr < h, rr, 3 * h - rr)
    iot3 = jnp.where(rr == h, h, iot3)
    wrows = []
    prows = []
    neg = jnp.float32(-jnp.inf)
    m = _red8(_Max, corr3)
    for i in range(_TOPK):
        idx = _red8(_Min, jnp.where(corr3 == m[None], iot3, _L))
        sel = iot3 == idx[None]
        pat = _red8(_Sum, jnp.where(sel, vb3, 0.0))
        wrows.append(m[0:1] + dc)
        prows.append(pat[0:1])
        if i + 1 < _TOPK:
            corr3 = jnp.where(sel, neg, corr3)
            m = _red8(_Max, corr3)
    zero = jnp.zeros_like(wrows[0])
    w_ref[...] = jnp.concatenate(wrows + [zero], axis=0)
    p_ref[...] = jnp.concatenate(prows + [zero], axis=0)


def _main_kernel(grh_ref, grl_ref, gih_ref, gil_ref,
                 q_ref, qr_ref, k_ref, kr_ref, v_ref,
                 w_ref, p_ref, corr_scr, dc_scr):
    j = pl.program_id(0)
    parity = jax.lax.rem(j, 2)
    omp = 1 - parity
    _topk(corr_scr.at[omp], dc_scr.at[omp], v_ref[...], w_ref, p_ref)
    _synth(grh_ref, grl_ref, gih_ref, gil_ref,
           q_ref[...], qr_ref[...], k_ref[...], kr_ref[...],
           corr_scr.at[parity], dc_scr.at[parity])


def _outer_kernel(p_ref, w_ref, o_ref):
    ph, plo = _split_f32(p_ref[...])
    wh, wl = _split_f32(w_ref[...])
    o_ref[...] = _dot3_t(ph, plo, wh, wl)


def kernel(Q, K, V):
    q0 = Q[0]
    k0 = K[0]
    v0 = V[0]
    h = _L // 2
    qrev = jnp.concatenate([q0[0:1], jnp.flip(q0[1:], axis=0)], axis=0)
    krev = jnp.concatenate([k0[0:1], jnp.flip(k0[1:], axis=0)], axis=0)
    vperm = jnp.concatenate(
        [v0[:h + 1], jnp.flip(v0[h + 1:], axis=0)], axis=0)
    grh = jnp.asarray(_GRH).astype(jnp.bfloat16)
    grl = jnp.asarray(_GRL).astype(jnp.bfloat16)
    gih = jnp.asarray(_GIH).astype(jnp.bfloat16)
    gil = jnp.asarray(_GIL).astype(jnp.bfloat16)

    full = pl.BlockSpec((_L // 2, _L // 2), lambda j: (0, 0))
    cur = pl.BlockSpec((_L, _BN), lambda j: (0, jnp.minimum(j, _NBLK - 1)))
    lag = pl.BlockSpec((_L, _BN), lambda j: (0, jnp.maximum(j - 1, 0)))
    lag8 = pl.BlockSpec((8, _BN), lambda j: (0, jnp.maximum(j - 1, 0)))

    wt, pt = pl.pallas_call(
        _main_kernel,
        grid=(_NBLK + 1,),
        in_specs=[full, full, full, full, cur, cur, cur, cur, lag],
        out_specs=[lag8, lag8],
        out_shape=[
            jax.ShapeDtypeStruct((8, _L), jnp.float32),
            jax.ShapeDtypeStruct((8, _L), jnp.float32),
        ],
        scratch_shapes=[
            pltpu.VMEM((2, _L, _BN), jnp.float32),
            pltpu.VMEM((2, 8, _BN), jnp.float32),
        ],
        compiler_params=pltpu.CompilerParams(
            vmem_limit_bytes=64 * 1024 * 1024),
    )(grh, grl, gih, gil, q0, qrev, k0, krev, vperm)

    out = pl.pallas_call(
        _outer_kernel,
        grid=(_L // _BM,),
        in_specs=[
            pl.BlockSpec((8, _BM), lambda i: (0, i)),
            pl.BlockSpec((8, _L), lambda i: (0, 0)),
        ],
        out_specs=pl.BlockSpec((_BM, _L), lambda i: (i, 0)),
        out_shape=jax.ShapeDtypeStruct((_L, _L), jnp.float32),
    )(pt, wt)
    return out[None]


# speedup vs baseline: 2.4314x; 2.2709x over previous
import numpy as np
import jax
import jax.numpy as jnp
from jax.experimental import pallas as pl
from jax.experimental.pallas import tpu as pltpu

_L = 2048
_TOPK = 7
_BN = 256
_BM = 256
_NBLK = _L // _BN


def _build_dft_consts():
    t = np.arange(_L, dtype=np.float64)
    om = np.arange(1, _L // 2 + 1, dtype=np.float64)
    th = 2.0 * np.pi * np.outer(om, t) / _L
    gr = np.cos(th)
    gi = -np.sin(th)
    return gr.astype(np.float32), gi.astype(np.float32)


def _split_hi_lo(a):
    import ml_dtypes
    hi32 = a.astype(ml_dtypes.bfloat16).astype(np.float32)
    lo32 = a - hi32
    return hi32, lo32


_GR_NP, _GI_NP = _build_dft_consts()
_GRH, _GRL = _split_hi_lo(_GR_NP)
_GIH, _GIL = _split_hi_lo(_GI_NP)


def _split_f32(x):
    hi = x.astype(jnp.bfloat16)
    lo = (x - hi.astype(jnp.float32)).astype(jnp.bfloat16)
    return hi, lo


def _dot3(ah, al, bh, bl):
    acc = jnp.dot(ah, bh, preferred_element_type=jnp.float32)
    acc += jnp.dot(ah, bl, preferred_element_type=jnp.float32)
    acc += jnp.dot(al, bh, preferred_element_type=jnp.float32)
    return acc


def _dot3_t(ah, al, bh, bl):
    dn = (((0,), (0,)), ((), ()))
    acc = jax.lax.dot_general(ah, bh, dn, preferred_element_type=jnp.float32)
    acc += jax.lax.dot_general(ah, bl, dn, preferred_element_type=jnp.float32)
    acc += jax.lax.dot_general(al, bh, dn, preferred_element_type=jnp.float32)
    return acc


def _synth(grh_ref, grl_ref, gih_ref, gil_ref, qb, kb, corr_ref, dc_ref):
    h = _L // 2
    qh, ql = _split_f32(qb)
    kh, kl = _split_f32(kb)
    qr = _dot3(grh_ref[...], grl_ref[...], qh, ql)
    qi = _dot3(gih_ref[...], gil_ref[...], qh, ql)
    kr = _dot3(grh_ref[...], grl_ref[...], kh, kl)
    ki = _dot3(gih_ref[...], gil_ref[...], kh, kl)
    iot_h = jax.lax.broadcasted_iota(jnp.int32, (h, _BN), 0)
    sc_re = jnp.where(iot_h == h - 1, jnp.float32(1.0 / _L),
                      jnp.float32(2.0 / _L))
    yre = (qr * kr + qi * ki) * sc_re
    yim = (qi * kr - qr * ki) * jnp.float32(2.0 / _L)
    yrh, yrl = _split_f32(yre)
    yih, yil = _split_f32(yim)
    corr = _dot3_t(grh_ref[...], grl_ref[...], yrh, yrl)
    corr += _dot3_t(gih_ref[...], gil_ref[...], yih, yil)
    corr_ref[...] = corr
    qs = jnp.sum(qb, axis=0, keepdims=True)
    ks = jnp.sum(kb, axis=0, keepdims=True)
    dc_ref[...] = jnp.broadcast_to(qs * ks * (1.0 / _L), (8, _BN))


def _topk(corr_ref, dc_ref, vb, w_ref, p_ref):
    dc = dc_ref[0:1, :]
    iot = jax.lax.broadcasted_iota(jnp.int32, (_L, _BN), 0)
    wrows = []
    prows = []
    neg = jnp.float32(-jnp.inf)
    for i in range(_TOPK):
        corr_p = corr_ref[...]
        m = jnp.max(corr_p, axis=0, keepdims=True)
        idx = jnp.min(jnp.where(corr_p == m, iot, _L), axis=0, keepdims=True)
        sel = iot == idx
        pat = jnp.sum(jnp.where(sel, vb, 0.0), axis=0, keepdims=True)
        wrows.append(m + dc)
        prows.append(pat)
        if i + 1 < _TOPK:
            corr_ref[...] = jnp.where(sel, neg, corr_p)
    zero = jnp.zeros_like(wrows[0])
    w_ref[...] = jnp.concatenate(wrows + [zero], axis=0)
    p_ref[...] = jnp.concatenate(prows + [zero], axis=0)


def _main_kernel(grh_ref, grl_ref, gih_ref, gil_ref, q_ref, k_ref, v_ref,
                 w_ref, p_ref, corr_scr, dc_scr):
    j = pl.program_id(0)
    parity = jax.lax.rem(j, 2)
    omp = 1 - parity
    _topk(corr_scr.at[omp], dc_scr.at[omp], v_ref[...], w_ref, p_ref)
    _synth(grh_ref, grl_ref, gih_ref, gil_ref, q_ref[...], k_ref[...],
           corr_scr.at[parity], dc_scr.at[parity])


def _outer_kernel(p_ref, w_ref, o_ref):
    ph, plo = _split_f32(p_ref[...])
    wh, wl = _split_f32(w_ref[...])
    o_ref[...] = _dot3_t(ph, plo, wh, wl)


def kernel(Q, K, V):
    q0 = Q[0]
    k0 = K[0]
    v0 = V[0]
    grh = jnp.asarray(_GRH).astype(jnp.bfloat16)
    grl = jnp.asarray(_GRL).astype(jnp.bfloat16)
    gih = jnp.asarray(_GIH).astype(jnp.bfloat16)
    gil = jnp.asarray(_GIL).astype(jnp.bfloat16)

    full = pl.BlockSpec((_L // 2, _L), lambda j: (0, 0))
    cur = pl.BlockSpec((_L, _BN), lambda j: (0, jnp.minimum(j, _NBLK - 1)))
    lag = pl.BlockSpec((_L, _BN), lambda j: (0, jnp.maximum(j - 1, 0)))
    lag8 = pl.BlockSpec((8, _BN), lambda j: (0, jnp.maximum(j - 1, 0)))

    wt, pt = pl.pallas_call(
        _main_kernel,
        grid=(_NBLK + 1,),
        in_specs=[full, full, full, full, cur, cur, lag],
        out_specs=[lag8, lag8],
        out_shape=[
            jax.ShapeDtypeStruct((8, _L), jnp.float32),
            jax.ShapeDtypeStruct((8, _L), jnp.float32),
        ],
        scratch_shapes=[
            pltpu.VMEM((2, _L, _BN), jnp.float32),
            pltpu.VMEM((2, 8, _BN), jnp.float32),
        ],
        compiler_params=pltpu.CompilerParams(
            vmem_limit_bytes=64 * 1024 * 1024),
    )(grh, grl, gih, gil, q0, k0, v0)

    out = pl.pallas_call(
        _outer_kernel,
        grid=(_L // _BM,),
        in_specs=[
            pl.BlockSpec((8, _BM), lambda i: (0, i)),
            pl.BlockSpec((8, _L), lambda i: (0, 0)),
        ],
        out_specs=pl.BlockSpec((_BM, _L), lambda i: (i, 0)),
        out_shape=jax.ShapeDtypeStruct((_L, _L), jnp.float32),
    )(pt, wt)
    return out[None]
